# Initial kernel scaffold; baseline (speedup 1.0000x reference)
import functools

import jax
import jax.numpy as jnp
from jax import lax
from jax.experimental import pallas as pl
from jax.experimental.pallas import tpu as pltpu
from jax.experimental.pallas import tpu_sc as plsc

NODES = 10000
PAD = 10240
EDGES = 320000
D = 128
NC = 2
NS = 16
NW = NC * NS
EPW = EDGES // NW
K = 80
CHUNKS = EPW // K
ZR = 128
RPT = PAD // NS
BR = 1024


def _make_sc_agg(with_deg):
    mesh = plsc.VectorSubcoreMesh(core_axis_name="c", subcore_axis_name="s")
    out_type = [jax.ShapeDtypeStruct((NC, PAD, D), jnp.float32)]
    scratch = [
        pltpu.VMEM((K,), jnp.int32),
        pltpu.VMEM((K,), jnp.int32),
        pltpu.VMEM((K, D), jnp.float32),
        pltpu.VMEM((ZR, D), jnp.float32),
        pltpu.VMEM_SHARED((PAD, D), jnp.float32),
        pltpu.SemaphoreType.DMA,
    ]
    if with_deg:
        out_type.append(jax.ShapeDtypeStruct((NW, PAD), jnp.float32))
        scratch.append(pltpu.VMEM((PAD,), jnp.float32))

    def body(x_hbm, src_hbm, dst_hbm, out_hbm, *rest):
        if with_deg:
            degp_hbm, sidx, didx, rows, zbuf, acc, sem, deg_v = rest
        else:
            sidx, didx, rows, zbuf, acc, sem = rest
        c = lax.axis_index("c")
        s = lax.axis_index("s")
        wid = s * NC + c
        zero16 = jnp.zeros((16,), jnp.float32)

        def zero_zbuf(i, carry):
            for j in range(D // 16):
                zbuf[i, pl.ds(j * 16, 16)] = zero16
            return carry

        lax.fori_loop(0, ZR, zero_zbuf, 0)
        for kk in range(RPT // ZR):
            pltpu.sync_copy(zbuf, acc.at[pl.ds(s * RPT + kk * ZR, ZR)])
        if with_deg:
            def zero_deg(i, carry):
                deg_v[pl.ds(i * 16, 16)] = zero16
                return carry
            lax.fori_loop(0, PAD // 16, zero_deg, 0)
        plsc.subcore_barrier()

        ones16 = jnp.full((16,), 1.0, jnp.float32)
        base0 = wid * EPW

        def chunk(ci, carry):
            b = base0 + ci * K
            pltpu.sync_copy(src_hbm.at[pl.ds(b, K)], sidx)
            pltpu.sync_copy(dst_hbm.at[pl.ds(b, K)], didx)
            pltpu.async_copy(x_hbm.at[sidx], rows, sem).wait()
            pltpu.sync_copy(rows, acc.at[didx], add=True)
            if with_deg:
                for j in range(K // 16):
                    idx = didx[pl.ds(j * 16, 16)]
                    plsc.addupdate_scatter(deg_v, [idx], ones16)
            return carry

        lax.fori_loop(0, CHUNKS, chunk, 0)
        plsc.subcore_barrier()
        pltpu.sync_copy(acc.at[pl.ds(s * RPT, RPT)],
                        out_hbm.at[c, pl.ds(s * RPT, RPT)])
        if with_deg:
            pltpu.sync_copy(deg_v, degp_hbm.at[wid])

    return functools.partial(
        pl.kernel, mesh=mesh, out_type=tuple(out_type),
        scratch_types=tuple(scratch))(body)


def _mm_body(x_ref, w_ref, o_ref):
    o_ref[...] = jnp.dot(x_ref[...], w_ref[...],
                         preferred_element_type=jnp.float32)


def _mm(x, w):
    return pl.pallas_call(
        _mm_body,
        grid=(PAD // BR,),
        in_specs=[pl.BlockSpec((BR, D), lambda i: (i, 0)),
                  pl.BlockSpec((D, D), lambda i: (0, 0))],
        out_specs=pl.BlockSpec((BR, D), lambda i: (i, 0)),
        out_shape=jax.ShapeDtypeStruct((PAD, D), jnp.float32),
    )(x, w)


def _combine_body(h_ref, p_ref, degt_ref, ws_ref, b_ref, wn_ref,
                  o1_ref, o2_ref):
    deg = jnp.sum(degt_ref[...], axis=1, keepdims=True)
    inv = 1.0 / jnp.maximum(deg, 1.0)
    agg = (p_ref[0] + p_ref[1]) * inv
    t = jnp.dot(h_ref[...], ws_ref[...],
                preferred_element_type=jnp.float32) + b_ref[...] + agg
    hr = jnp.maximum(t, 0.0)
    o1_ref[...] = hr
    o2_ref[...] = jnp.dot(hr, wn_ref[...],
                          preferred_element_type=jnp.float32)


def _combine(h, p, degt, ws, b, wn):
    return pl.pallas_call(
        _combine_body,
        grid=(PAD // BR,),
        in_specs=[pl.BlockSpec((BR, D), lambda i: (i, 0)),
                  pl.BlockSpec((NC, BR, D), lambda i: (0, i, 0)),
                  pl.BlockSpec((BR, NW), lambda i: (i, 0)),
                  pl.BlockSpec((D, D), lambda i: (0, 0)),
                  pl.BlockSpec((1, D), lambda i: (0, 0)),
                  pl.BlockSpec((D, D), lambda i: (0, 0))],
        out_specs=[pl.BlockSpec((BR, D), lambda i: (i, 0)),
                   pl.BlockSpec((BR, D), lambda i: (i, 0))],
        out_shape=[jax.ShapeDtypeStruct((PAD, D), jnp.float32),
                   jax.ShapeDtypeStruct((PAD, D), jnp.float32)],
    )(h, p, degt, ws, b, wn)


def _final_body(h_ref, p_ref, degt_ref, ws_ref, b_ref, o_ref):
    deg = jnp.sum(degt_ref[...], axis=1, keepdims=True)
    inv = 1.0 / jnp.maximum(deg, 1.0)
    agg = (p_ref[0] + p_ref[1]) * inv
    o_ref[...] = jnp.dot(h_ref[...], ws_ref[...],
                         preferred_element_type=jnp.float32) + b_ref[...] + agg


def _final(h, p, degt, ws, b):
    return pl.pallas_call(
        _final_body,
        grid=(PAD // BR,),
        in_specs=[pl.BlockSpec((BR, D), lambda i: (i, 0)),
                  pl.BlockSpec((NC, BR, D), lambda i: (0, i, 0)),
                  pl.BlockSpec((BR, NW), lambda i: (i, 0)),
                  pl.BlockSpec((D, D), lambda i: (0, 0)),
                  pl.BlockSpec((1, D), lambda i: (0, 0))],
        out_specs=pl.BlockSpec((BR, D), lambda i: (i, 0)),
        out_shape=jax.ShapeDtypeStruct((PAD, D), jnp.float32),
    )(h, p, degt, ws, b)


def kernel(h, edge_index, W_self0, W_neigh0, b0, W_self1, W_neigh1, b1,
           W_self2, W_neigh2, b2):
    src = edge_index[0]
    dst = edge_index[1]
    h_pad = jnp.pad(h, ((0, PAD - NODES), (0, 0)))
    b0r = b0.reshape(1, D)
    b1r = b1.reshape(1, D)
    b2r = b2.reshape(1, D)

    sc_agg_deg = _make_sc_agg(True)
    sc_agg = _make_sc_agg(False)

    hn0 = _mm(h_pad, W_neigh0)
    p0, degp = sc_agg_deg(hn0, src, dst)
    degt = degp.T
    h1, hn1 = _combine(h_pad, p0, degt, W_self0, b0r, W_neigh1)
    (p1,) = sc_agg(hn1, src, dst)
    h2, hn2 = _combine(h1, p1, degt, W_self1, b1r, W_neigh2)
    (p2,) = sc_agg(hn2, src, dst)
    out = _final(h2, p2, degt, W_self2, b2r)
    return out[:NODES]


# speedup vs baseline: 5.2841x; 5.2841x over previous
import functools

import jax
import jax.numpy as jnp
from jax import lax
from jax.experimental import pallas as pl
from jax.experimental.pallas import tpu as pltpu
from jax.experimental.pallas import tpu_sc as plsc

NODES = 10000
PAD = 10240
EDGES = 320000
D = 128
NC = 2
NS = 16
NW = NC * NS
EPW = EDGES // NW
K = 80
CHUNKS = EPW // K
ZR = 128
RPT = PAD // NS
BR = 1024


def _make_sc_agg(with_deg):
    mesh = plsc.VectorSubcoreMesh(core_axis_name="c", subcore_axis_name="s")
    out_type = [jax.ShapeDtypeStruct((NC, PAD, D), jnp.float32)]
    scratch = [
        pltpu.VMEM((K,), jnp.int32),
        pltpu.VMEM((K,), jnp.int32),
        pltpu.VMEM((K, D), jnp.float32),
        pltpu.VMEM((ZR, D), jnp.float32),
        pltpu.VMEM_SHARED((PAD, D), jnp.float32),
        pltpu.SemaphoreType.DMA,
    ]
    if with_deg:
        out_type.append(jax.ShapeDtypeStruct((NW, PAD), jnp.float32))
        scratch.append(pltpu.VMEM((PAD,), jnp.float32))

    def body(x_hbm, src_hbm, dst_hbm, out_hbm, *rest):
        if with_deg:
            degp_hbm, sidx, didx, rows, zbuf, acc, sem, deg_v = rest
        else:
            sidx, didx, rows, zbuf, acc, sem = rest
        c = lax.axis_index("c")
        s = lax.axis_index("s")
        wid = s * NC + c
        zero16 = jnp.zeros((16,), jnp.float32)

        def zero_zbuf(i, carry):
            for j in range(D // 16):
                zbuf[i, pl.ds(j * 16, 16)] = zero16
            return carry

        lax.fori_loop(0, ZR, zero_zbuf, 0)
        for kk in range(RPT // ZR):
            pltpu.sync_copy(zbuf, acc.at[pl.ds(s * RPT + kk * ZR, ZR)])
        if with_deg:
            def zero_deg(i, carry):
                deg_v[pl.ds(i * 16, 16)] = zero16
                return carry
            lax.fori_loop(0, PAD // 16, zero_deg, 0)
        plsc.subcore_barrier()

        ones16 = jnp.full((16,), 1.0, jnp.float32)
        base0 = wid * EPW

        def chunk(ci, carry):
            b = base0 + ci * K
            pltpu.sync_copy(src_hbm.at[pl.ds(b, K)], sidx)
            pltpu.sync_copy(dst_hbm.at[pl.ds(b, K)], didx)
            pltpu.async_copy(x_hbm.at[sidx], rows, sem).wait()
            pltpu.sync_copy(rows, acc.at[didx], add=True)
            if with_deg:
                for j in range(K // 16):
                    idx = didx[pl.ds(j * 16, 16)]
                    plsc.addupdate_scatter(deg_v, [idx], ones16)
            return carry

        lax.fori_loop(0, CHUNKS, chunk, 0)
        plsc.subcore_barrier()
        pltpu.sync_copy(acc.at[pl.ds(s * RPT, RPT)],
                        out_hbm.at[c, pl.ds(s * RPT, RPT)])
        if with_deg:
            pltpu.sync_copy(deg_v, degp_hbm.at[wid])

    return functools.partial(
        pl.kernel, mesh=mesh, out_type=tuple(out_type),
        scratch_types=tuple(scratch),
        compiler_params=pltpu.CompilerParams(needs_layout_passes=False))(body)


def _mm_body(x_ref, w_ref, o_ref):
    o_ref[...] = jnp.dot(x_ref[...], w_ref[...],
                         preferred_element_type=jnp.float32)


def _mm(x, w):
    return pl.pallas_call(
        _mm_body,
        grid=(PAD // BR,),
        in_specs=[pl.BlockSpec((BR, D), lambda i: (i, 0)),
                  pl.BlockSpec((D, D), lambda i: (0, 0))],
        out_specs=pl.BlockSpec((BR, D), lambda i: (i, 0)),
        out_shape=jax.ShapeDtypeStruct((PAD, D), jnp.float32),
    )(x, w)


def _combine_body(h_ref, p_ref, degt_ref, ws_ref, b_ref, wn_ref,
                  o1_ref, o2_ref):
    deg = jnp.sum(degt_ref[...], axis=1, keepdims=True)
    inv = 1.0 / jnp.maximum(deg, 1.0)
    agg = (p_ref[0] + p_ref[1]) * inv
    t = jnp.dot(h_ref[...], ws_ref[...],
                preferred_element_type=jnp.float32) + b_ref[...] + agg
    hr = jnp.maximum(t, 0.0)
    o1_ref[...] = hr
    o2_ref[...] = jnp.dot(hr, wn_ref[...],
                          preferred_element_type=jnp.float32)


def _combine(h, p, degt, ws, b, wn):
    return pl.pallas_call(
        _combine_body,
        grid=(PAD // BR,),
        in_specs=[pl.BlockSpec((BR, D), lambda i: (i, 0)),
                  pl.BlockSpec((NC, BR, D), lambda i: (0, i, 0)),
                  pl.BlockSpec((BR, NW), lambda i: (i, 0)),
                  pl.BlockSpec((D, D), lambda i: (0, 0)),
                  pl.BlockSpec((1, D), lambda i: (0, 0)),
                  pl.BlockSpec((D, D), lambda i: (0, 0))],
        out_specs=[pl.BlockSpec((BR, D), lambda i: (i, 0)),
                   pl.BlockSpec((BR, D), lambda i: (i, 0))],
        out_shape=[jax.ShapeDtypeStruct((PAD, D), jnp.float32),
                   jax.ShapeDtypeStruct((PAD, D), jnp.float32)],
    )(h, p, degt, ws, b, wn)


def _final_body(h_ref, p_ref, degt_ref, ws_ref, b_ref, o_ref):
    deg = jnp.sum(degt_ref[...], axis=1, keepdims=True)
    inv = 1.0 / jnp.maximum(deg, 1.0)
    agg = (p_ref[0] + p_ref[1]) * inv
    o_ref[...] = jnp.dot(h_ref[...], ws_ref[...],
                         preferred_element_type=jnp.float32) + b_ref[...] + agg


def _final(h, p, degt, ws, b):
    return pl.pallas_call(
        _final_body,
        grid=(PAD // BR,),
        in_specs=[pl.BlockSpec((BR, D), lambda i: (i, 0)),
                  pl.BlockSpec((NC, BR, D), lambda i: (0, i, 0)),
                  pl.BlockSpec((BR, NW), lambda i: (i, 0)),
                  pl.BlockSpec((D, D), lambda i: (0, 0)),
                  pl.BlockSpec((1, D), lambda i: (0, 0))],
        out_specs=pl.BlockSpec((BR, D), lambda i: (i, 0)),
        out_shape=jax.ShapeDtypeStruct((PAD, D), jnp.float32),
    )(h, p, degt, ws, b)


def kernel(h, edge_index, W_self0, W_neigh0, b0, W_self1, W_neigh1, b1,
           W_self2, W_neigh2, b2):
    src = edge_index[0]
    dst = edge_index[1]
    h_pad = jnp.pad(h, ((0, PAD - NODES), (0, 0)))
    b0r = b0.reshape(1, D)
    b1r = b1.reshape(1, D)
    b2r = b2.reshape(1, D)

    sc_agg_deg = _make_sc_agg(True)
    sc_agg = _make_sc_agg(False)

    hn0 = _mm(h_pad, W_neigh0)
    p0, degp = sc_agg_deg(hn0, src, dst)
    degt = degp.T
    h1, hn1 = _combine(h_pad, p0, degt, W_self0, b0r, W_neigh1)
    (p1,) = sc_agg(hn1, src, dst)
    h2, hn2 = _combine(h1, p1, degt, W_self1, b1r, W_neigh2)
    (p2,) = sc_agg(hn2, src, dst)
    out = _final(h2, p2, degt, W_self2, b2r)
    return out[:NODES]
